# Initial kernel scaffold; baseline (speedup 1.0000x reference)
import jax
import jax.numpy as jnp
from jax.experimental import pallas as pl


def kernel(p, plane_weights, mlp_W1, mlp_b1, bn1_g, bn1_b, mlp_W2, mlp_b2, bn2_g, bn2_b, mlp_W3, mlp_b3, bn3_g, bn3_b, mlp_W4, mlp_b4, blk_fc0_W, blk_fc0_b, blk_fc1_W, blk_fc1_b, blk_sc_W, fc_c_W, fc_c_b):
    raise NotImplementedError("write your pallas kernel here")


# speedup vs baseline: 1.7685x; 1.7685x over previous
import functools
import math

import jax
import jax.numpy as jnp
from jax import lax
from jax.experimental import pallas as pl
from jax.experimental.pallas import tpu as pltpu

B, T, D = 2, 2048, 3
CH, K, H = 3, 40, 512
KP1 = K + 1
LN_BASE = math.log(2.73)

RT = 256
MT = 512
BT = 512


def _knn_feats_kernel(p_rows_ref, p_all_ref, w_ref, out_ref):
    p_rows = p_rows_ref[0]
    p_all = p_all_ref[0]

    sq_rows = jnp.sum(p_rows * p_rows, axis=1, keepdims=True)
    sq_all = jnp.sum(p_all * p_all, axis=1, keepdims=True)
    prod = lax.dot_general(p_rows.astype(jnp.bfloat16),
                           p_all.astype(jnp.bfloat16),
                           (((1,), (1,)), ((), ())),
                           preferred_element_type=jnp.float32)
    d2 = sq_rows + sq_all.T - 2.0 * prod

    iota = lax.broadcasted_iota(jnp.int32, (RT, T), 1)
    big_i = jnp.int32(T + 1)
    inf = jnp.float32(jnp.inf)

    row0 = pl.program_id(1) * RT
    riota = lax.broadcasted_iota(jnp.int32, (RT, T), 0)
    diag = iota == riota + row0

    w = w_ref[...]
    wvec = w[:, 1:]
    wn = jnp.sqrt(jnp.sum(wvec * wvec, axis=1))
    w_bf = w.astype(jnp.bfloat16)
    ones_col = jnp.ones((RT, 1), jnp.float32)

    def body(_, carry):
        d2c, acc = carry
        m = jnp.min(d2c, axis=1, keepdims=True)
        is_min = d2c == m
        first = jnp.min(jnp.where(is_min, iota, big_i), axis=1, keepdims=True)
        pick = iota == first
        pick_f = pick.astype(jnp.float32)
        nbr = lax.dot_general(pick_f, p_all, (((1,), (0,)), ((), ())),
                              preferred_element_type=jnp.float32,
                              precision=lax.Precision.HIGHEST)
        diff = nbr - p_rows
        aug = jnp.concatenate([ones_col, diff], axis=1)
        h = lax.dot_general(aug.astype(jnp.bfloat16), w_bf,
                            (((1,), (1,)), ((), ())),
                            preferred_element_type=jnp.float32)
        h = h / wn[None, :]
        notself = 1.0 - jnp.sum(jnp.where(diag, pick_f, 0.0),
                                axis=1, keepdims=True)
        acc = acc + h * notself
        d2c = jnp.where(pick, inf, d2c)
        return d2c, acc

    acc0 = jnp.zeros((RT, CH), jnp.float32)
    _, acc = lax.fori_loop(0, KP1, body, (d2, acc0))

    pc = acc / jnp.float32(K)
    out_ref[0] = 1.0 / (1.0 + jnp.exp(pc * jnp.float32(LN_BASE)))


def _knn_feats(p, plane_weights):
    grid = (B, T // RT)
    return pl.pallas_call(
        _knn_feats_kernel,
        grid=grid,
        in_specs=[
            pl.BlockSpec((1, RT, D), lambda b, i: (b, i, 0)),
            pl.BlockSpec((1, T, D), lambda b, i: (b, 0, 0)),
            pl.BlockSpec((CH, 4), lambda b, i: (0, 0)),
        ],
        out_specs=pl.BlockSpec((1, RT, CH), lambda b, i: (b, i, 0)),
        out_shape=jax.ShapeDtypeStruct((B, T, CH), jnp.float32),
    )(p, p, plane_weights)


def _bn2(y0, y1, g, b):
    m = 0.5 * (y0 + y1)
    d0 = y0 - m
    d1 = y1 - m
    v = 0.5 * (d0 * d0 + d1 * d1)
    sd = jnp.sqrt(v + 1e-5)
    return (g * d0) / sd + b, (g * d1) / sd + b


def _mm_bf(a, w):
    return lax.dot_general(a.astype(jnp.bfloat16), w.astype(jnp.bfloat16),
                           (((1,), (0,)), ((), ())),
                           preferred_element_type=jnp.float32)


def _mlp_kernel(f_ref, W1, b1, g1, bb1, W2, b2, g2, bb2, W3, b3, g3, bb3,
                W4, b4, out_ref):
    def dense(x, W, b):
        return _mm_bf(x, W[...]) + b[...][None, :]

    x0 = f_ref[0]
    x1 = f_ref[1]
    y0, y1 = _bn2(dense(x0, W1, b1), dense(x1, W1, b1), g1[...], bb1[...])
    x0, x1 = jnp.maximum(y0, 0.0), jnp.maximum(y1, 0.0)
    y0, y1 = _bn2(dense(x0, W2, b2), dense(x1, W2, b2), g2[...], bb2[...])
    x0, x1 = jnp.maximum(y0, 0.0), jnp.maximum(y1, 0.0)
    y0, y1 = _bn2(dense(x0, W3, b3), dense(x1, W3, b3), g3[...], bb3[...])
    x0, x1 = jnp.maximum(y0, 0.0), jnp.maximum(y1, 0.0)
    out_ref[0] = dense(x0, W4, b4)
    out_ref[1] = dense(x1, W4, b4)


def _mlp(feats, W1, b1, g1, bb1, W2, b2, g2, bb2, W3, b3, g3, bb3, W4, b4):
    full = lambda s: pl.BlockSpec(s, lambda i: tuple(0 for _ in s))
    return pl.pallas_call(
        _mlp_kernel,
        grid=(T // MT,),
        in_specs=[
            pl.BlockSpec((B, MT, CH), lambda i: (0, i, 0)),
            full((CH, 64)), full((64,)), full((64,)), full((64,)),
            full((64, 128)), full((128,)), full((128,)), full((128,)),
            full((128, 128)), full((128,)), full((128,)), full((128,)),
            full((128, 2 * H)), full((2 * H,)),
        ],
        out_specs=pl.BlockSpec((B, MT, 2 * H), lambda i: (0, i, 0)),
        out_shape=jax.ShapeDtypeStruct((B, T, 2 * H), jnp.float32),
    )(feats, W1, b1, g1, bb1, W2, b2, g2, bb2, W3, b3, g3, bb3, W4, b4)


def _block_body(x, W0, b0, W1, b1, Ws):
    net = _mm_bf(jnp.maximum(x, 0.0), W0[...]) + b0[...][None, :]
    dx = _mm_bf(jnp.maximum(net, 0.0), W1[...]) + b1[...][None, :]
    return _mm_bf(x, Ws[...]) + dx


def _block0_kernel(net_ref, W0, b0, W1, b1, Ws, out_ref, max_ref, macc):
    j = pl.program_id(0)

    @pl.when(j == 0)
    def _():
        macc[...] = jnp.full((B, H), -jnp.inf, jnp.float32)

    for b in range(B):
        o = _block_body(net_ref[b], W0, b0, W1, b1, Ws)
        out_ref[b] = o
        macc[b, :] = jnp.maximum(macc[b, :], jnp.max(o, axis=0))

    @pl.when(j == T // BT - 1)
    def _():
        max_ref[...] = macc[...]


def _blockn_kernel(net_ref, pooled_ref, W0, b0, W1, b1, Ws, out_ref, max_ref,
                   macc, *, last, fc_W=None, fc_b=None):
    j = pl.program_id(0)

    @pl.when(j == 0)
    def _():
        macc[...] = jnp.full((B, H), -jnp.inf, jnp.float32)

    for b in range(B):
        pooled = jnp.broadcast_to(pooled_ref[b][None, :], (BT, H))
        x = jnp.concatenate([net_ref[b], pooled], axis=1)
        o = _block_body(x, W0, b0, W1, b1, Ws)
        out_ref[b] = o
        macc[b, :] = jnp.maximum(macc[b, :], jnp.max(o, axis=0))

    if last:
        @pl.when(j == T // BT - 1)
        def _():
            r = jnp.maximum(macc[...], 0.0)
            max_ref[...] = _mm_bf(r, fc_W[...]) + fc_b[...][None, :]
    else:
        @pl.when(j == T // BT - 1)
        def _():
            max_ref[...] = macc[...]


def _run_block0(net, W0, b0, W1, b1, Ws):
    full = lambda s: pl.BlockSpec(s, lambda i: tuple(0 for _ in s))
    return pl.pallas_call(
        _block0_kernel,
        grid=(T // BT,),
        in_specs=[
            pl.BlockSpec((B, BT, 2 * H), lambda i: (0, i, 0)),
            full((2 * H, H)), full((H,)), full((H, H)), full((H,)),
            full((2 * H, H)),
        ],
        out_specs=[
            pl.BlockSpec((B, BT, H), lambda i: (0, i, 0)),
            full((B, H)),
        ],
        out_shape=[
            jax.ShapeDtypeStruct((B, T, H), jnp.float32),
            jax.ShapeDtypeStruct((B, H), jnp.float32),
        ],
        scratch_shapes=[pltpu.VMEM((B, H), jnp.float32)],
    )(net, W0, b0, W1, b1, Ws)


def _run_blockn(net, pooled, W0, b0, W1, b1, Ws, fc_W=None, fc_b=None):
    last = fc_W is not None
    full = lambda s: pl.BlockSpec(s, lambda i: tuple(0 for _ in s))
    in_specs = [
        pl.BlockSpec((B, BT, H), lambda i: (0, i, 0)),
        full((B, H)),
        full((2 * H, H)), full((H,)), full((H, H)), full((H,)),
        full((2 * H, H)),
    ]
    args = [net, pooled, W0, b0, W1, b1, Ws]
    if last:
        def kern_last(net_ref, pooled_ref, W0r, b0r, W1r, b1r, Wsr, fcW, fcb,
                      out_ref, max_ref, macc):
            _blockn_kernel(net_ref, pooled_ref, W0r, b0r, W1r, b1r, Wsr,
                           out_ref, max_ref, macc, last=True, fc_W=fcW, fc_b=fcb)

        body = kern_last
        in_specs += [full((H, H)), full((H,))]
        args += [fc_W, fc_b]
    else:
        body = functools.partial(_blockn_kernel, last=False)
    return pl.pallas_call(
        body,
        grid=(T // BT,),
        in_specs=in_specs,
        out_specs=[
            pl.BlockSpec((B, BT, H), lambda i: (0, i, 0)),
            full((B, H)),
        ],
        out_shape=[
            jax.ShapeDtypeStruct((B, T, H), jnp.float32),
            jax.ShapeDtypeStruct((B, H), jnp.float32),
        ],
        scratch_shapes=[pltpu.VMEM((B, H), jnp.float32)],
    )(*args)


def kernel(p, plane_weights, mlp_W1, mlp_b1, bn1_g, bn1_b, mlp_W2, mlp_b2,
           bn2_g, bn2_b, mlp_W3, mlp_b3, bn3_g, bn3_b, mlp_W4, mlp_b4,
           blk_fc0_W, blk_fc0_b, blk_fc1_W, blk_fc1_b, blk_sc_W, fc_c_W, fc_c_b):
    feats = _knn_feats(p, plane_weights)
    net0 = _mlp(feats, mlp_W1, mlp_b1, bn1_g, bn1_b, mlp_W2, mlp_b2, bn2_g,
                bn2_b, mlp_W3, mlp_b3, bn3_g, bn3_b, mlp_W4, mlp_b4)
    net, pooled = _run_block0(net0, blk_fc0_W[0], blk_fc0_b[0], blk_fc1_W[0],
                              blk_fc1_b[0], blk_sc_W[0])
    for i in range(1, 4):
        net, pooled = _run_blockn(net, pooled, blk_fc0_W[i], blk_fc0_b[i],
                                  blk_fc1_W[i], blk_fc1_b[i], blk_sc_W[i])
    _, out = _run_blockn(net, pooled, blk_fc0_W[4], blk_fc0_b[4],
                         blk_fc1_W[4], blk_fc1_b[4], blk_sc_W[4],
                         fc_c_W, fc_c_b)
    return out
